# baseline (speedup 1.0000x reference)
import jax
import jax.numpy as jnp
from jax import lax
from jax.experimental import pallas as pl
from jax.experimental.pallas import tpu as pltpu
from jax.experimental.pallas import tpu_sc as plsc

D_MODEL = 1024
N_EXPERTS = 8
FFN_HIDDEN = 2048
TOKENS = 2048
CAPACITY = 512
SLOTS = N_EXPERTS * CAPACITY
BUF_ROWS = SLOTS + 8
YB_ROWS = SLOTS + CAPACITY

SC_CORES = 2
SC_SUBCORES = 16
NW = SC_CORES * SC_SUBCORES
TPW = TOKENS // NW

FT = 512
NF = FFN_HIDDEN // FT
RB = 128
NR = CAPACITY // RB


def _route_body(x_ref, wr_ref, sslot_ref, gslot_ref, scale_ref, cnt_ref):
    x = x_ref[...]
    wr = wr_ref[...]
    logits = jnp.dot(x, wr, preferred_element_type=jnp.float32)
    cost = jnp.exp(logits)
    eps = 1e-8
    d0 = jnp.ones((TOKENS, 1), jnp.float32)
    d1 = jnp.ones((1, N_EXPERTS), jnp.float32)
    for _ in range(3):
        d0 = (1.0 / TOKENS) / (jnp.sum(d1 * cost, axis=1, keepdims=True) + eps)
        d1 = (1.0 / N_EXPERTS) / (jnp.sum(d0 * cost, axis=0, keepdims=True) + eps)
    route = d1 * cost * d0
    m = jnp.max(route, axis=1, keepdims=True)
    col = lax.broadcasted_iota(jnp.int32, (TOKENS, N_EXPERTS), 1)
    eidx = jnp.min(jnp.where(route == m, col, N_EXPERTS), axis=1, keepdims=True)
    onehot = (col == eidx).astype(jnp.float32)
    gate = jnp.sum(jax.nn.sigmoid(logits) * onehot, axis=1, keepdims=True)
    cum = onehot
    s = 1
    while s < TOKENS:
        cum = cum + jnp.concatenate(
            [jnp.zeros((s, N_EXPERTS), jnp.float32), cum[:-s]], axis=0)
        s *= 2
    pos = jnp.sum(cum * onehot, axis=1, keepdims=True).astype(jnp.int32) - 1
    within = pos < CAPACITY
    slot = eidx * CAPACITY + pos
    tok = lax.broadcasted_iota(jnp.int32, (TOKENS, 1), 0)
    sslot_ref[...] = jnp.where(within, slot, SLOTS + (tok % 8)).reshape(TOKENS)
    gslot_ref[...] = jnp.where(
        within, slot, SLOTS + (tok % CAPACITY)).reshape(TOKENS)
    scale_ref[...] = jnp.broadcast_to(jnp.where(within, gate, 0.0), (TOKENS, 128))
    cnt_ref[...] = cum[TOKENS - 1:TOKENS, :].astype(jnp.int32)


def _route(x, Wr):
    return pl.pallas_call(
        _route_body,
        out_shape=(
            jax.ShapeDtypeStruct((TOKENS,), jnp.int32),
            jax.ShapeDtypeStruct((TOKENS,), jnp.int32),
            jax.ShapeDtypeStruct((TOKENS, 128), jnp.float32),
            jax.ShapeDtypeStruct((1, N_EXPERTS), jnp.int32),
        ),
    )(x, Wr)


def _sc_mesh():
    return plsc.VectorSubcoreMesh(core_axis_name="c", subcore_axis_name="s")


def _dispatch_body(x_hbm, slot_hbm, scale16_hbm, buf_hbm, sscale_hbm,
                   idx_v, rows_v, s16_v, sem):
    wid = lax.axis_index("s") * SC_CORES + lax.axis_index("c")
    base = wid * TPW
    pltpu.sync_copy(slot_hbm.at[pl.ds(base, TPW)], idx_v)
    pltpu.sync_copy(x_hbm.at[pl.ds(base, TPW)], rows_v)
    pltpu.sync_copy(scale16_hbm.at[pl.ds(base, TPW)], s16_v)
    c1 = pltpu.async_copy(rows_v, buf_hbm.at[idx_v], sem)
    pltpu.async_copy(s16_v, sscale_hbm.at[idx_v], sem).wait()
    c1.wait()


def _dispatch(x, sslot, scale16):
    return pl.kernel(
        _dispatch_body,
        out_type=(
            jax.ShapeDtypeStruct((BUF_ROWS, D_MODEL), jnp.float32),
            jax.ShapeDtypeStruct((BUF_ROWS, 128), jnp.float32),
        ),
        mesh=_sc_mesh(),
        scratch_types=[
            pltpu.VMEM((TPW,), jnp.int32),
            pltpu.VMEM((TPW, D_MODEL), jnp.float32),
            pltpu.VMEM((TPW, 128), jnp.float32),
            pltpu.SemaphoreType.DMA,
        ],
    )(x, sslot, scale16)


def _mlp_body(cnt_ref, buf_ref, w1g_ref, w1u_ref, w2_ref, ss_ref, out_ref):
    e = pl.program_id(0)
    j = pl.program_id(1)

    @pl.when(e == N_EXPERTS)
    def _():
        @pl.when(j == 0)
        def _():
            out_ref[...] = jnp.zeros((CAPACITY, D_MODEL), jnp.float32)

    @pl.when(e < N_EXPERTS)
    def _():
        cnt = cnt_ref[0, e]
        w1g = w1g_ref[0].astype(jnp.bfloat16)
        w1u = w1u_ref[0].astype(jnp.bfloat16)
        w2 = w2_ref[0].astype(jnp.bfloat16)
        for r in range(NR):
            @pl.when(cnt > r * RB)
            def _():
                rows = pl.ds(r * RB, RB)
                a = buf_ref[rows, :].astype(jnp.bfloat16)
                g = jnp.dot(a, w1g, preferred_element_type=jnp.float32)
                u = jnp.dot(a, w1u, preferred_element_type=jnp.float32)
                act = (g * jax.nn.sigmoid(g) * u).astype(jnp.bfloat16)
                yp = jnp.dot(act, w2, preferred_element_type=jnp.float32)
                if NF == 1:
                    out_ref[rows, :] = yp * ss_ref[rows, 0:1]
                else:
                    @pl.when(j == 0)
                    def _():
                        out_ref[rows, :] = yp

                    @pl.when(jnp.logical_and(j > 0, j < NF - 1))
                    def _():
                        out_ref[rows, :] += yp

                    @pl.when(j == NF - 1)
                    def _():
                        out_ref[rows, :] = (
                            (out_ref[rows, :] + yp) * ss_ref[rows, 0:1])


def _mlp(cnt, buf, W1, W2, sscale):
    grid = (N_EXPERTS + 1, NF)
    spec_buf = pl.BlockSpec(
        (CAPACITY, D_MODEL),
        lambda e, j, cnt: (jnp.where(e == N_EXPERTS, N_EXPERTS - 1, e), 0))
    spec_ss = pl.BlockSpec(
        (CAPACITY, 128),
        lambda e, j, cnt: (jnp.where(e == N_EXPERTS, N_EXPERTS - 1, e), 0))

    def wmap(col_off):
        def f(e, j, cnt):
            last = e == N_EXPERTS
            return (jnp.where(last, N_EXPERTS - 1, e), 0,
                    jnp.where(last, NF - 1, j) + col_off)
        return f

    spec_w1g = pl.BlockSpec((1, D_MODEL, FT), wmap(0))
    spec_w1u = pl.BlockSpec((1, D_MODEL, FT), wmap(NF))
    spec_w2 = pl.BlockSpec(
        (1, FT, D_MODEL),
        lambda e, j, cnt: (jnp.where(e == N_EXPERTS, N_EXPERTS - 1, e),
                           jnp.where(e == N_EXPERTS, NF - 1, j), 0))
    return pl.pallas_call(
        _mlp_body,
        grid_spec=pltpu.PrefetchScalarGridSpec(
            num_scalar_prefetch=1,
            grid=grid,
            in_specs=[spec_buf, spec_w1g, spec_w1u, spec_w2, spec_ss],
            out_specs=pl.BlockSpec((CAPACITY, D_MODEL),
                                   lambda e, j, cnt: (e, 0)),
        ),
        out_shape=jax.ShapeDtypeStruct((YB_ROWS, D_MODEL), jnp.float32),
        compiler_params=pltpu.CompilerParams(
            dimension_semantics=("arbitrary", "arbitrary")),
    )(cnt, buf, W1, W1, W2, sscale)


def _combine_body(yb_hbm, gslot_hbm, y_hbm, idx_v, rows_v, sem):
    wid = lax.axis_index("s") * SC_CORES + lax.axis_index("c")
    base = wid * TPW
    pltpu.sync_copy(gslot_hbm.at[pl.ds(base, TPW)], idx_v)
    pltpu.async_copy(yb_hbm.at[idx_v], rows_v, sem).wait()
    pltpu.sync_copy(rows_v, y_hbm.at[pl.ds(base, TPW)])


def _combine(yb, gslot):
    return pl.kernel(
        _combine_body,
        out_type=jax.ShapeDtypeStruct((TOKENS, D_MODEL), jnp.float32),
        mesh=_sc_mesh(),
        scratch_types=[
            pltpu.VMEM((TPW,), jnp.int32),
            pltpu.VMEM((TPW, D_MODEL), jnp.float32),
            pltpu.SemaphoreType.DMA,
        ],
    )(yb, gslot)


def kernel(x, Wr, W1, W2):
    sslot, gslot, scale16, cnt = _route(x, Wr)
    buf, sscale = _dispatch(x, sslot, scale16)
    yb = _mlp(cnt, buf, W1, W2, sscale)
    return _combine(yb, gslot)


# speedup vs baseline: 1.1152x; 1.1152x over previous
import jax
import jax.numpy as jnp
from jax import lax
from jax.experimental import pallas as pl
from jax.experimental.pallas import tpu as pltpu
from jax.experimental.pallas import tpu_sc as plsc

D_MODEL = 1024
N_EXPERTS = 8
FFN_HIDDEN = 2048
TOKENS = 2048
CAPACITY = 512
SLOTS = N_EXPERTS * CAPACITY
BUF_ROWS = SLOTS + 8
YB_ROWS = SLOTS + CAPACITY

SC_CORES = 2
SC_SUBCORES = 16
NW = SC_CORES * SC_SUBCORES
TPW = TOKENS // NW

FT = 1024
NF = FFN_HIDDEN // FT
RB = 128
NR = CAPACITY // RB


RTB = 256
NRT = TOKENS // RTB


def _route_body(x_ref, wr_ref, sslot_ref, gslot_ref, scale_ref, cnt_ref,
                lg_ref):
    i = pl.program_id(0)
    lg_ref[pl.ds(i * RTB, RTB), :] = jnp.dot(
        x_ref[...], wr_ref[...], preferred_element_type=jnp.float32)

    @pl.when(i == NRT - 1)
    def _():
        _route_finish(lg_ref, sslot_ref, gslot_ref, scale_ref, cnt_ref)


def _route_finish(lg_ref, sslot_ref, gslot_ref, scale_ref, cnt_ref):
    logits = lg_ref[...]
    cost = jnp.exp(logits)
    eps = 1e-8
    d0 = jnp.ones((TOKENS, 1), jnp.float32)
    d1 = jnp.ones((1, N_EXPERTS), jnp.float32)
    for _ in range(3):
        d0 = (1.0 / TOKENS) / (jnp.sum(d1 * cost, axis=1, keepdims=True) + eps)
        d1 = (1.0 / N_EXPERTS) / (jnp.sum(d0 * cost, axis=0, keepdims=True) + eps)
    route = d1 * cost * d0
    m = jnp.max(route, axis=1, keepdims=True)
    col = lax.broadcasted_iota(jnp.int32, (TOKENS, N_EXPERTS), 1)
    eidx = jnp.min(jnp.where(route == m, col, N_EXPERTS), axis=1, keepdims=True)
    onehot = (col == eidx).astype(jnp.float32)
    gate = jnp.sum(jax.nn.sigmoid(logits) * onehot, axis=1, keepdims=True)
    cum = onehot
    s = 1
    while s < TOKENS:
        cum = cum + jnp.concatenate(
            [jnp.zeros((s, N_EXPERTS), jnp.float32), cum[:-s]], axis=0)
        s *= 2
    pos = jnp.sum(cum * onehot, axis=1, keepdims=True).astype(jnp.int32) - 1
    within = pos < CAPACITY
    slot = eidx * CAPACITY + pos
    tok = lax.broadcasted_iota(jnp.int32, (TOKENS, 1), 0)
    sslot_ref[...] = jnp.where(within, slot, SLOTS + (tok % 8)).reshape(TOKENS)
    gslot_ref[...] = jnp.where(
        within, slot, SLOTS + (tok % CAPACITY)).reshape(TOKENS)
    scale_ref[...] = jnp.broadcast_to(jnp.where(within, gate, 0.0), (TOKENS, 128))
    cnt_ref[...] = cum[TOKENS - 1:TOKENS, :].astype(jnp.int32)


def _route(x, Wr):
    return pl.pallas_call(
        _route_body,
        grid=(NRT,),
        in_specs=[
            pl.BlockSpec((RTB, D_MODEL), lambda i: (i, 0)),
            pl.BlockSpec((D_MODEL, N_EXPERTS), lambda i: (0, 0)),
        ],
        out_specs=(
            pl.BlockSpec((TOKENS,), lambda i: (0,)),
            pl.BlockSpec((TOKENS,), lambda i: (0,)),
            pl.BlockSpec((TOKENS, 128), lambda i: (0, 0)),
            pl.BlockSpec((1, N_EXPERTS), lambda i: (0, 0)),
        ),
        out_shape=(
            jax.ShapeDtypeStruct((TOKENS,), jnp.int32),
            jax.ShapeDtypeStruct((TOKENS,), jnp.int32),
            jax.ShapeDtypeStruct((TOKENS, 128), jnp.float32),
            jax.ShapeDtypeStruct((1, N_EXPERTS), jnp.int32),
        ),
        scratch_shapes=[pltpu.VMEM((TOKENS, N_EXPERTS), jnp.float32)],
        compiler_params=pltpu.CompilerParams(
            dimension_semantics=("arbitrary",)),
    )(x, Wr)


def _sc_mesh():
    return plsc.VectorSubcoreMesh(core_axis_name="c", subcore_axis_name="s")


def _dispatch_body(x_hbm, slot_hbm, scale16_hbm, buf_hbm, sscale_hbm,
                   idx_v, rows_v, s16_v, sem):
    wid = lax.axis_index("s") * SC_CORES + lax.axis_index("c")
    base = wid * TPW
    r1 = pltpu.async_copy(slot_hbm.at[pl.ds(base, TPW)], idx_v, sem)
    r2 = pltpu.async_copy(x_hbm.at[pl.ds(base, TPW)], rows_v, sem)
    r3 = pltpu.async_copy(scale16_hbm.at[pl.ds(base, TPW)], s16_v, sem)
    r1.wait()
    r2.wait()
    r3.wait()
    c1 = pltpu.async_copy(rows_v, buf_hbm.at[idx_v], sem)
    pltpu.async_copy(s16_v, sscale_hbm.at[idx_v], sem).wait()
    c1.wait()


def _dispatch(x, sslot, scale16):
    return pl.kernel(
        _dispatch_body,
        out_type=(
            jax.ShapeDtypeStruct((BUF_ROWS, D_MODEL), jnp.float32),
            jax.ShapeDtypeStruct((BUF_ROWS, 128), jnp.float32),
        ),
        mesh=_sc_mesh(),
        scratch_types=[
            pltpu.VMEM((TPW,), jnp.int32),
            pltpu.VMEM((TPW, D_MODEL), jnp.float32),
            pltpu.VMEM((TPW, 128), jnp.float32),
            pltpu.SemaphoreType.DMA,
        ],
    )(x, sslot, scale16)


def _mlp_body(cnt_ref, buf_ref, w1g_ref, w1u_ref, w2_ref, ss_ref, out_ref):
    e = pl.program_id(0)
    j = pl.program_id(1)

    @pl.when(e == N_EXPERTS)
    def _():
        @pl.when(j == 0)
        def _():
            out_ref[...] = jnp.zeros((CAPACITY, D_MODEL), jnp.float32)

    @pl.when(e < N_EXPERTS)
    def _():
        cnt = cnt_ref[0, e]
        w1g = w1g_ref[0].astype(jnp.bfloat16)
        w1u = w1u_ref[0].astype(jnp.bfloat16)
        w2 = w2_ref[0].astype(jnp.bfloat16)
        for r in range(NR):
            @pl.when(cnt > r * RB)
            def _():
                rows = pl.ds(r * RB, RB)
                a = buf_ref[rows, :].astype(jnp.bfloat16)
                g = jnp.dot(a, w1g, preferred_element_type=jnp.float32)
                u = jnp.dot(a, w1u, preferred_element_type=jnp.float32)
                act = (g * jax.nn.sigmoid(g) * u).astype(jnp.bfloat16)
                yp = jnp.dot(act, w2, preferred_element_type=jnp.float32)
                if NF == 1:
                    out_ref[rows, :] = yp * ss_ref[rows, 0:1]
                else:
                    @pl.when(j == 0)
                    def _():
                        out_ref[rows, :] = yp

                    @pl.when(jnp.logical_and(j > 0, j < NF - 1))
                    def _():
                        out_ref[rows, :] += yp

                    @pl.when(j == NF - 1)
                    def _():
                        out_ref[rows, :] = (
                            (out_ref[rows, :] + yp) * ss_ref[rows, 0:1])


def _mlp(cnt, buf, W1, W2, sscale):
    grid = (N_EXPERTS + 1, NF)
    spec_buf = pl.BlockSpec(
        (CAPACITY, D_MODEL),
        lambda e, j, cnt: (jnp.where(e == N_EXPERTS, N_EXPERTS - 1, e), 0))
    spec_ss = pl.BlockSpec(
        (CAPACITY, 128),
        lambda e, j, cnt: (jnp.where(e == N_EXPERTS, N_EXPERTS - 1, e), 0))

    def wmap(col_off):
        def f(e, j, cnt):
            last = e == N_EXPERTS
            return (jnp.where(last, N_EXPERTS - 1, e), 0,
                    jnp.where(last, NF - 1, j) + col_off)
        return f

    spec_w1g = pl.BlockSpec((1, D_MODEL, FT), wmap(0))
    spec_w1u = pl.BlockSpec((1, D_MODEL, FT), wmap(NF))
    spec_w2 = pl.BlockSpec(
        (1, FT, D_MODEL),
        lambda e, j, cnt: (jnp.where(e == N_EXPERTS, N_EXPERTS - 1, e),
                           jnp.where(e == N_EXPERTS, NF - 1, j), 0))
    return pl.pallas_call(
        _mlp_body,
        grid_spec=pltpu.PrefetchScalarGridSpec(
            num_scalar_prefetch=1,
            grid=grid,
            in_specs=[spec_buf, spec_w1g, spec_w1u, spec_w2, spec_ss],
            out_specs=pl.BlockSpec((CAPACITY, D_MODEL),
                                   lambda e, j, cnt: (e, 0)),
        ),
        out_shape=jax.ShapeDtypeStruct((YB_ROWS, D_MODEL), jnp.float32),
        compiler_params=pltpu.CompilerParams(
            dimension_semantics=("arbitrary", "arbitrary")),
    )(cnt, buf, W1, W1, W2, sscale)


def _combine_body(yb_hbm, gslot_hbm, y_hbm, idx_v, rows_v, sem):
    wid = lax.axis_index("s") * SC_CORES + lax.axis_index("c")
    base = wid * TPW
    pltpu.sync_copy(gslot_hbm.at[pl.ds(base, TPW)], idx_v)
    pltpu.async_copy(yb_hbm.at[idx_v], rows_v, sem).wait()
    pltpu.sync_copy(rows_v, y_hbm.at[pl.ds(base, TPW)])


def _combine(yb, gslot):
    return pl.kernel(
        _combine_body,
        out_type=jax.ShapeDtypeStruct((TOKENS, D_MODEL), jnp.float32),
        mesh=_sc_mesh(),
        scratch_types=[
            pltpu.VMEM((TPW,), jnp.int32),
            pltpu.VMEM((TPW, D_MODEL), jnp.float32),
            pltpu.SemaphoreType.DMA,
        ],
    )(yb, gslot)


def kernel(x, Wr, W1, W2):
    sslot, gslot, scale16, cnt = _route(x, Wr)
    buf, sscale = _dispatch(x, sslot, scale16)
    yb = _mlp(cnt, buf, W1, W2, sscale)
    return _combine(yb, gslot)


# speedup vs baseline: 1.1388x; 1.0211x over previous
import jax
import jax.numpy as jnp
from jax import lax
from jax.experimental import pallas as pl
from jax.experimental.pallas import tpu as pltpu
from jax.experimental.pallas import tpu_sc as plsc

D_MODEL = 1024
N_EXPERTS = 8
FFN_HIDDEN = 2048
TOKENS = 2048
CAPACITY = 512
SLOTS = N_EXPERTS * CAPACITY
BUF_ROWS = SLOTS + 8
YB_ROWS = SLOTS + CAPACITY

SC_CORES = 2
SC_SUBCORES = 16
NW = SC_CORES * SC_SUBCORES
TPW = TOKENS // NW

FT = 1024
NF = FFN_HIDDEN // FT
RB = 128
NR = CAPACITY // RB


def _route_body(x_ref, wr_ref, sslot_ref, gslot_ref, scale_ref, cnt_ref):
    logits = jnp.dot(x_ref[...], wr_ref[...],
                     preferred_element_type=jnp.float32)
    cost = jnp.exp(logits)
    eps = 1e-8
    d0 = jnp.ones((TOKENS, 1), jnp.float32)
    d1 = jnp.ones((1, N_EXPERTS), jnp.float32)
    for _ in range(3):
        d0 = (1.0 / TOKENS) / (jnp.sum(d1 * cost, axis=1, keepdims=True) + eps)
        d1 = (1.0 / N_EXPERTS) / (jnp.sum(d0 * cost, axis=0, keepdims=True) + eps)
    route = d1 * cost * d0
    m = jnp.max(route, axis=1, keepdims=True)
    col = lax.broadcasted_iota(jnp.int32, (TOKENS, N_EXPERTS), 1)
    eidx = jnp.min(jnp.where(route == m, col, N_EXPERTS), axis=1, keepdims=True)
    onehot = (col == eidx).astype(jnp.float32)
    gate = jnp.sum(jax.nn.sigmoid(logits) * onehot, axis=1, keepdims=True)
    cum = onehot
    s = 1
    while s < TOKENS:
        cum = cum + jnp.concatenate(
            [jnp.zeros((s, N_EXPERTS), jnp.float32), cum[:-s]], axis=0)
        s *= 2
    pos = jnp.sum(cum * onehot, axis=1, keepdims=True).astype(jnp.int32) - 1
    within = pos < CAPACITY
    slot = eidx * CAPACITY + pos
    tok = lax.broadcasted_iota(jnp.int32, (TOKENS, 1), 0)
    sslot_ref[...] = jnp.where(within, slot, SLOTS + (tok % 8)).reshape(TOKENS)
    gslot_ref[...] = jnp.where(
        within, slot, SLOTS + (tok % CAPACITY)).reshape(TOKENS)
    scale_ref[...] = jnp.broadcast_to(jnp.where(within, gate, 0.0), (TOKENS, 128))
    cnt_ref[...] = cum[TOKENS - 1:TOKENS, :].astype(jnp.int32)


def _route(x, Wr):
    return pl.pallas_call(
        _route_body,
        out_shape=(
            jax.ShapeDtypeStruct((TOKENS,), jnp.int32),
            jax.ShapeDtypeStruct((TOKENS,), jnp.int32),
            jax.ShapeDtypeStruct((TOKENS, 128), jnp.float32),
            jax.ShapeDtypeStruct((1, N_EXPERTS), jnp.int32),
        ),
    )(x, Wr)


def _sc_mesh():
    return plsc.VectorSubcoreMesh(core_axis_name="c", subcore_axis_name="s")


def _dispatch_body(x_hbm, slot_hbm, scale16_hbm, buf_hbm, sscale_hbm,
                   idx_v, rows_v, s16_v, sem):
    wid = lax.axis_index("s") * SC_CORES + lax.axis_index("c")
    base = wid * TPW
    r1 = pltpu.async_copy(slot_hbm.at[pl.ds(base, TPW)], idx_v, sem)
    r2 = pltpu.async_copy(x_hbm.at[pl.ds(base, TPW)], rows_v, sem)
    r3 = pltpu.async_copy(scale16_hbm.at[pl.ds(base, TPW)], s16_v, sem)
    r1.wait()
    r2.wait()
    r3.wait()
    c1 = pltpu.async_copy(rows_v, buf_hbm.at[idx_v], sem)
    pltpu.async_copy(s16_v, sscale_hbm.at[idx_v], sem).wait()
    c1.wait()


def _dispatch(x, sslot, scale16):
    return pl.kernel(
        _dispatch_body,
        out_type=(
            jax.ShapeDtypeStruct((BUF_ROWS, D_MODEL), jnp.float32),
            jax.ShapeDtypeStruct((BUF_ROWS, 128), jnp.float32),
        ),
        mesh=_sc_mesh(),
        scratch_types=[
            pltpu.VMEM((TPW,), jnp.int32),
            pltpu.VMEM((TPW, D_MODEL), jnp.float32),
            pltpu.VMEM((TPW, 128), jnp.float32),
            pltpu.SemaphoreType.DMA,
        ],
    )(x, sslot, scale16)


def _mlp_body(cnt_ref, buf_ref, w1g_ref, w1u_ref, w2_ref, ss_ref, out_ref):
    e = pl.program_id(0)
    j = pl.program_id(1)

    @pl.when(e == N_EXPERTS)
    def _():
        @pl.when(j == 0)
        def _():
            out_ref[...] = jnp.zeros((CAPACITY, D_MODEL), jnp.float32)

    @pl.when(e < N_EXPERTS)
    def _():
        cnt = cnt_ref[0, e]
        w1g = w1g_ref[0].astype(jnp.bfloat16)
        w1u = w1u_ref[0].astype(jnp.bfloat16)
        w2 = w2_ref[0].astype(jnp.bfloat16)
        for r in range(NR):
            @pl.when(cnt > r * RB)
            def _():
                rows = pl.ds(r * RB, RB)
                a = buf_ref[rows, :].astype(jnp.bfloat16)
                g = jnp.dot(a, w1g, preferred_element_type=jnp.float32)
                u = jnp.dot(a, w1u, preferred_element_type=jnp.float32)
                act = (g * jax.nn.sigmoid(g) * u).astype(jnp.bfloat16)
                yp = jnp.dot(act, w2, preferred_element_type=jnp.float32)
                if NF == 1:
                    out_ref[rows, :] = yp * ss_ref[rows, 0:1]
                else:
                    @pl.when(j == 0)
                    def _():
                        out_ref[rows, :] = yp

                    @pl.when(jnp.logical_and(j > 0, j < NF - 1))
                    def _():
                        out_ref[rows, :] += yp

                    @pl.when(j == NF - 1)
                    def _():
                        out_ref[rows, :] = (
                            (out_ref[rows, :] + yp) * ss_ref[rows, 0:1])


def _mlp(cnt, buf, W1, W2, sscale):
    grid = (N_EXPERTS + 1, NF)
    spec_buf = pl.BlockSpec(
        (CAPACITY, D_MODEL),
        lambda e, j, cnt: (jnp.where(e == N_EXPERTS, N_EXPERTS - 1, e), 0))
    spec_ss = pl.BlockSpec(
        (CAPACITY, 128),
        lambda e, j, cnt: (jnp.where(e == N_EXPERTS, N_EXPERTS - 1, e), 0))

    def wmap(col_off):
        def f(e, j, cnt):
            last = e == N_EXPERTS
            return (jnp.where(last, N_EXPERTS - 1, e), 0,
                    jnp.where(last, NF - 1, j) + col_off)
        return f

    spec_w1g = pl.BlockSpec((1, D_MODEL, FT), wmap(0))
    spec_w1u = pl.BlockSpec((1, D_MODEL, FT), wmap(NF))
    spec_w2 = pl.BlockSpec(
        (1, FT, D_MODEL),
        lambda e, j, cnt: (jnp.where(e == N_EXPERTS, N_EXPERTS - 1, e),
                           jnp.where(e == N_EXPERTS, NF - 1, j), 0))
    return pl.pallas_call(
        _mlp_body,
        grid_spec=pltpu.PrefetchScalarGridSpec(
            num_scalar_prefetch=1,
            grid=grid,
            in_specs=[spec_buf, spec_w1g, spec_w1u, spec_w2, spec_ss],
            out_specs=pl.BlockSpec((CAPACITY, D_MODEL),
                                   lambda e, j, cnt: (e, 0)),
        ),
        out_shape=jax.ShapeDtypeStruct((YB_ROWS, D_MODEL), jnp.float32),
        compiler_params=pltpu.CompilerParams(
            dimension_semantics=("arbitrary", "arbitrary")),
    )(cnt, buf, W1, W1, W2, sscale)


def _combine_body(yb_hbm, gslot_hbm, y_hbm, idx_v, rows_v, sem):
    wid = lax.axis_index("s") * SC_CORES + lax.axis_index("c")
    base = wid * TPW
    pltpu.sync_copy(gslot_hbm.at[pl.ds(base, TPW)], idx_v)
    pltpu.async_copy(yb_hbm.at[idx_v], rows_v, sem).wait()
    pltpu.sync_copy(rows_v, y_hbm.at[pl.ds(base, TPW)])


def _combine(yb, gslot):
    return pl.kernel(
        _combine_body,
        out_type=jax.ShapeDtypeStruct((TOKENS, D_MODEL), jnp.float32),
        mesh=_sc_mesh(),
        scratch_types=[
            pltpu.VMEM((TPW,), jnp.int32),
            pltpu.VMEM((TPW, D_MODEL), jnp.float32),
            pltpu.SemaphoreType.DMA,
        ],
    )(yb, gslot)


def kernel(x, Wr, W1, W2):
    sslot, gslot, scale16, cnt = _route(x, Wr)
    buf, sscale = _dispatch(x, sslot, scale16)
    yb = _mlp(cnt, buf, W1, W2, sscale)
    return _combine(yb, gslot)
